# Initial kernel scaffold; baseline (speedup 1.0000x reference)
import jax
import jax.numpy as jnp
from jax.experimental import pallas as pl


def kernel(x, edge_index, batch, edge_attr, W_rel1, b_rel1, W_root1, W_rel3, b_rel3, W_root3, W_lin, b_lin):
    raise NotImplementedError("write your pallas kernel here")


# speedup vs baseline: 3.4084x; 3.4084x over previous
import functools

import jax
import jax.numpy as jnp
from jax import lax
from jax.experimental import pallas as pl
from jax.experimental.pallas import tpu as pltpu
from jax.experimental.pallas import tpu_sc as plsc

NUM_CORES = 2
NUM_SUBCORES = 16
LANES = 16
NW = NUM_CORES * NUM_SUBCORES

D = 128
FV = D // LANES
CHUNK = 128


def _seg_sum_sc(feat, src, dst, w, n_pad):
  e_pad = src.shape[0]
  ept = e_pad // NW
  nchunks = ept // CHUNK
  zchunks = n_pad // CHUNK // NUM_SUBCORES

  mesh = plsc.VectorSubcoreMesh(core_axis_name="c", subcore_axis_name="s")

  @functools.partial(
      pl.kernel,
      out_type=jax.ShapeDtypeStruct((NUM_CORES, n_pad, D), jnp.float32),
      mesh=mesh,
      scratch_types=[
          pltpu.VMEM_SHARED((n_pad, D), jnp.float32),
          pltpu.VMEM((1, CHUNK), jnp.int32),
          pltpu.VMEM((1, CHUNK), jnp.int32),
          pltpu.VMEM((1, CHUNK), jnp.float32),
          pltpu.VMEM((1, CHUNK, D), jnp.float32),
          pltpu.SemaphoreType.DMA,
      ],
  )
  def seg_kernel(feat_hbm, src_hbm, dst_hbm, w_hbm, out_hbm,
                 acc, sidx, didx, wbuf, rows, gsem):
    c = lax.axis_index("c")
    s = lax.axis_index("s")

    @pl.loop(0, CHUNK)
    def _zero_rows(i):
      for f in range(FV):
        rows[0, i, pl.ds(f * LANES, LANES)] = jnp.zeros((LANES,), jnp.float32)

    for z in range(zchunks):
      r0 = (s * zchunks + z) * CHUNK
      pltpu.sync_copy(rows.at[0], acc.at[pl.ds(r0, CHUNK)])
    plsc.subcore_barrier()

    tbase = (c * NUM_SUBCORES + s) * ept

    @pl.loop(0, nchunks)
    def _edge_chunk(k):
      base = tbase + k * CHUNK
      pltpu.sync_copy(src_hbm.at[pl.ds(base, CHUNK)], sidx.at[0])
      pltpu.sync_copy(dst_hbm.at[pl.ds(base, CHUNK)], didx.at[0])
      pltpu.sync_copy(w_hbm.at[pl.ds(base, CHUNK)], wbuf.at[0])
      pltpu.async_copy(feat_hbm.at[sidx.at[0]], rows.at[0], gsem).wait()

      @pl.loop(0, CHUNK // LANES)
      def _scale_group(g):
        wv = wbuf[0, pl.ds(g * LANES, LANES)]
        for j in range(LANES):
          wj = lax.gather(
              wv, jnp.full((LANES, 1), j, jnp.int32),
              lax.GatherDimensionNumbers(offset_dims=(),
                                         collapsed_slice_dims=(0,),
                                         start_index_map=(0,)),
              slice_sizes=(1,),
              mode=lax.GatherScatterMode.PROMISE_IN_BOUNDS)
          i = g * LANES + j
          for f in range(FV):
            sl = pl.ds(f * LANES, LANES)
            rows[0, i, sl] = rows[0, i, sl] * wj

      pltpu.sync_copy(rows.at[0], acc.at[didx.at[0]], add=True)

    plsc.subcore_barrier()

    for z in range(zchunks):
      r0 = (s * zchunks + z) * CHUNK
      pltpu.sync_copy(acc.at[pl.ds(r0, CHUNK)], out_hbm.at[c, pl.ds(r0, CHUNK)])

  return seg_kernel(feat, src, dst, w)


def _layer_tc(p0, p1, x, wrelT, brel, wrootT, block_n):
  n = x.shape[0]
  grid = n // block_n

  def body(a_ref, b_ref, x_ref, wr_ref, br_ref, wt_ref, o_ref):
    agg = a_ref[...] + b_ref[...]
    acc = jnp.dot(agg, wr_ref[...], preferred_element_type=jnp.float32)
    acc += jnp.dot(x_ref[...], wt_ref[...], preferred_element_type=jnp.float32)
    o_ref[...] = jnp.maximum(acc + br_ref[...], 0.0)

  return pl.pallas_call(
      body,
      grid=(grid,),
      in_specs=[
          pl.BlockSpec((block_n, D), lambda i: (i, 0)),
          pl.BlockSpec((block_n, D), lambda i: (i, 0)),
          pl.BlockSpec((block_n, D), lambda i: (i, 0)),
          pl.BlockSpec((D, D), lambda i: (0, 0)),
          pl.BlockSpec((1, D), lambda i: (0, 0)),
          pl.BlockSpec((D, D), lambda i: (0, 0)),
      ],
      out_specs=pl.BlockSpec((block_n, D), lambda i: (i, 0)),
      out_shape=jax.ShapeDtypeStruct((n, D), jnp.float32),
  )(p0, p1, x, wrelT, brel, wrootT)


def _final_tc(p0, p1, h, batch3, wrelT, brel, wrootT, wlin, blin, block_n, g):
  n = h.shape[0]
  grid = n // block_n

  def body(a_ref, b_ref, h_ref, bt_ref, wr_ref, br_ref, wt_ref,
           wl_ref, bl_ref, o_ref, sums, counts):
    i = pl.program_id(0)

    @pl.when(i == 0)
    def _():
      sums[...] = jnp.zeros_like(sums)
      counts[...] = jnp.zeros_like(counts)

    agg = a_ref[...] + b_ref[...]
    h2 = jnp.dot(agg, wr_ref[...], preferred_element_type=jnp.float32)
    h2 += jnp.dot(h_ref[...], wt_ref[...], preferred_element_type=jnp.float32)
    h2 += br_ref[...]

    bvec = bt_ref[0, 0, :]
    onehot = (bvec[:, None] == lax.broadcasted_iota(jnp.int32, (1, g), 1)
              ).astype(jnp.float32)
    sums[...] += lax.dot_general(onehot, h2, (((0,), (0,)), ((), ())),
                                 preferred_element_type=jnp.float32)
    counts[...] += lax.dot_general(
        onehot, jnp.ones((block_n, D), jnp.float32), (((0,), (0,)), ((), ())),
        preferred_element_type=jnp.float32)

    @pl.when(i == pl.num_programs(0) - 1)
    def _():
      pooled = sums[...] / jnp.maximum(counts[...], 1.0)
      val = jnp.sum(pooled * wl_ref[...], axis=1, keepdims=True)
      o_ref[...] = jnp.maximum(val + bl_ref[...], 0.0) * jnp.ones((g, D),
                                                                  jnp.float32)

  return pl.pallas_call(
      body,
      grid=(grid,),
      in_specs=[
          pl.BlockSpec((block_n, D), lambda i: (i, 0)),
          pl.BlockSpec((block_n, D), lambda i: (i, 0)),
          pl.BlockSpec((block_n, D), lambda i: (i, 0)),
          pl.BlockSpec((1, 1, block_n), lambda i: (i, 0, 0)),
          pl.BlockSpec((D, D), lambda i: (0, 0)),
          pl.BlockSpec((1, D), lambda i: (0, 0)),
          pl.BlockSpec((D, D), lambda i: (0, 0)),
          pl.BlockSpec((1, D), lambda i: (0, 0)),
          pl.BlockSpec((1, 1), lambda i: (0, 0)),
      ],
      out_specs=pl.BlockSpec((g, D), lambda i: (0, 0)),
      out_shape=jax.ShapeDtypeStruct((g, D), jnp.float32),
      scratch_shapes=[
          pltpu.VMEM((g, D), jnp.float32),
          pltpu.VMEM((g, D), jnp.float32),
      ],
  )(p0, p1, h, batch3, wrelT, brel, wrootT, wlin, blin)


def kernel(x, edge_index, batch, edge_attr, W_rel1, b_rel1, W_root1,
           W_rel3, b_rel3, W_root3, W_lin, b_lin):
  n, d = x.shape
  e = edge_attr.shape[0]
  g = int(jnp.ndim(W_lin) and W_lin.shape[0]) or 1
  num_graphs = 64

  ept = -(-e // (NW * CHUNK)) * CHUNK
  e_pad = ept * NW
  pad = e_pad - e
  src = jnp.pad(edge_index[0], (0, pad))
  dst = jnp.pad(edge_index[1], (0, pad))
  w = jnp.pad(edge_attr, (0, pad))

  n_pad = -(-n // (CHUNK * NUM_SUBCORES)) * (CHUNK * NUM_SUBCORES)

  block_n = 2000
  batch3 = batch.reshape(n // block_n, 1, block_n)

  agg1 = _seg_sum_sc(x, src, dst, w, n_pad)
  h = _layer_tc(agg1[0, :n], agg1[1, :n], x, W_rel1.T,
                b_rel1.reshape(1, d), W_root1.T, block_n)
  agg2 = _seg_sum_sc(h, src, dst, w, n_pad)
  outf = _final_tc(agg2[0, :n], agg2[1, :n], h, batch3, W_rel3.T,
                   b_rel3.reshape(1, d), W_root3.T, W_lin,
                   b_lin.reshape(1, 1), block_n, num_graphs)
  return outf[:, :1]
